# baseline (speedup 1.0000x reference)
import functools

import jax
import jax.numpy as jnp
from jax import lax
from jax.experimental import pallas as pl
from jax.experimental.pallas import tpu as pltpu


def _sig(v):
    return 0.5 * jnp.tanh(0.5 * v) + 0.5


def _stacked_kernel(x_ref, trial_ref,
                    w0p_ref, b0p_ref, whf0_ref, whb0_ref,
                    w1p_ref, b1p_ref, whf1_ref, whb1_ref,
                    w2p_ref, b2p_ref, whf2_ref, whb2_ref,
                    w0a_ref, w0b_ref, b0_ref, midw_ref, midb_ref,
                    wl_ref, bl_ref,
                    out_rnn_ref, out_log_ref,
                    xg, seq0, seq1, *, T, B, H, unroll):
    G = 4 * H
    GG = 8 * H
    f32 = jnp.float32
    bf16 = jnp.bfloat16

    def cell(g, c_prev):
        i_g = _sig(g[:, 0 * H:1 * H])
        f_g = _sig(g[:, 1 * H:2 * H])
        g_g = jnp.tanh(g[:, 2 * H:3 * H])
        o_g = _sig(g[:, 3 * H:4 * H])
        c = f_g * c_prev + i_g * g_g
        h = o_g * jnp.tanh(c)
        return h, c

    def cell0(g):
        i_g = _sig(g[:, 0 * H:1 * H])
        g_g = jnp.tanh(g[:, 2 * H:3 * H])
        o_g = _sig(g[:, 3 * H:4 * H])
        c = i_g * g_g
        h = o_g * jnp.tanh(c)
        return h, c

    def run_layer(inp_ref, wp_ref, bp_ref, whf_ref, whb_ref, seq_out, l):
        last = seq_out is None

        if not last:
            xg[...] = (
                jnp.dot(inp_ref[...], wp_ref[...],
                        preferred_element_type=f32)
                + bp_ref[...]).astype(bf16)
        else:
            xg[:, 0:G] = (
                jnp.dot(inp_ref[...], wp_ref[:, 0:G],
                        preferred_element_type=f32)
                + bp_ref[:, 0:G]).astype(bf16)
            xg[(T - 1) * B:T * B, G:GG] = (
                jnp.dot(inp_ref[(T - 1) * B:T * B, :], wp_ref[:, G:GG],
                        preferred_element_type=f32)
                + bp_ref[:, G:GG]).astype(bf16)

        g0f = xg[0:B, 0:G].astype(f32)
        g0b = xg[(T - 1) * B:T * B, G:GG].astype(f32)
        hf, cf = cell0(g0f)
        hb, cb = cell0(g0b)
        out_rnn_ref[:, l * 2 * H + H:l * 2 * H + 2 * H] = hb
        if not last:
            seq_out[0:B, 0:H] = hf.astype(bf16)
            seq_out[(T - 1) * B:T * B, H:2 * H] = hb.astype(bf16)

        steps_left = T - 1
        u = next(c for c in (unroll, 9, 7, 3, 1) if steps_left % c == 0)
        n_outer = steps_left // u

        def one_step(t, hf, cf, hb, cb):
            rf = pl.multiple_of(t * B, 16)
            gf = xg[pl.ds(rf, B), 0:G].astype(f32) + jnp.dot(
                hf.astype(bf16), whf_ref[...], preferred_element_type=f32)
            hf, cf = cell(gf, cf)
            if not last:
                rb = pl.multiple_of((T - 1 - t) * B, 16)
                gb = xg[pl.ds(rb, B), G:GG].astype(f32) + jnp.dot(
                    hb.astype(bf16), whb_ref[...], preferred_element_type=f32)
                hb, cb = cell(gb, cb)
                seq_out[pl.ds(rf, B), 0:H] = hf.astype(bf16)
                seq_out[pl.ds(rb, B), H:2 * H] = hb.astype(bf16)
            return hf, cf, hb, cb

        if n_outer == 1:
            carry = (hf, cf, hb, cb)
            for t in range(1, T):
                carry = one_step(t, *carry)
            hf, cf, hb, cb = carry
        else:
            def outer(k, carry):
                t0 = 1 + k * u
                for uu in range(u):
                    carry = one_step(t0 + uu, *carry)
                return carry

            hf, cf, hb, cb = lax.fori_loop(0, n_outer, outer,
                                           (hf, cf, hb, cb))
        out_rnn_ref[:, l * 2 * H:l * 2 * H + H] = hf

    run_layer(x_ref, w0p_ref, b0p_ref, whf0_ref, whb0_ref, seq0, 0)
    run_layer(seq0, w1p_ref, b1p_ref, whf1_ref, whb1_ref, seq1, 1)
    run_layer(seq1, w2p_ref, b2p_ref, whf2_ref, whb2_ref, None, 2)

    rnn = out_rnn_ref[...]
    h0 = jnp.maximum(
        jnp.dot(rnn, w0a_ref[...], preferred_element_type=f32)
        + jnp.dot(trial_ref[...], w0b_ref[...], preferred_element_type=f32)
        + b0_ref[...], 0.0)
    h1 = jnp.maximum(
        jnp.dot(h0, midw_ref[...], preferred_element_type=f32)
        + midb_ref[...], 0.0)
    out_log_ref[...] = (
        jnp.dot(h1, wl_ref[...], preferred_element_type=f32) + bl_ref[...])


def kernel(x, trial_vec,
           rnn0_w_proj, rnn0_b_proj, rnn0_whh_f, rnn0_whh_b,
           rnn1_w_proj, rnn1_b_proj, rnn1_whh_f, rnn1_whh_b,
           rnn2_w_proj, rnn2_b_proj, rnn2_whh_f, rnn2_whh_b,
           fc_w0a, fc_w0b, fc_b0, fc_mid0_w, fc_mid0_b, fc_wl, fc_bl):
    B, T, D = x.shape
    H = rnn0_whh_f.shape[0]
    M = T * B
    n_class = fc_bl.shape[-1]
    n_rnn_feat = fc_w0a.shape[0]

    x2 = jnp.transpose(x.astype(jnp.bfloat16), (1, 0, 2)).reshape(M, D)
    trial = trial_vec.astype(jnp.float32)

    vspec = pl.BlockSpec(memory_space=pltpu.MemorySpace.VMEM)
    out_rnn, out_log = pl.pallas_call(
        functools.partial(_stacked_kernel, T=T, B=B, H=H, unroll=63),
        out_shape=(
            jax.ShapeDtypeStruct((B, n_rnn_feat), jnp.float32),
            jax.ShapeDtypeStruct((B, n_class), jnp.float32),
        ),
        in_specs=[vspec] * 21,
        out_specs=(vspec, vspec),
        scratch_shapes=[
            pltpu.VMEM((M, 8 * H), jnp.bfloat16),
            pltpu.VMEM((M, 2 * H), jnp.bfloat16),
            pltpu.VMEM((M, 2 * H), jnp.bfloat16),
        ],
        compiler_params=pltpu.CompilerParams(
            vmem_limit_bytes=56 * 1024 * 1024,
        ),
    )(x2, trial,
      rnn0_w_proj, rnn0_b_proj, rnn0_whh_f, rnn0_whh_b,
      rnn1_w_proj, rnn1_b_proj, rnn1_whh_f, rnn1_whh_b,
      rnn2_w_proj, rnn2_b_proj, rnn2_whh_f, rnn2_whh_b,
      fc_w0a, fc_w0b, fc_b0, fc_mid0_w, fc_mid0_b, fc_wl, fc_bl)
    return out_rnn, out_log
